# Initial kernel scaffold; baseline (speedup 1.0000x reference)
import jax
import jax.numpy as jnp
from jax.experimental import pallas as pl


def kernel(x, router_w, fc1_w, fc1_b, fc2_w, fc2_b):
    raise NotImplementedError("write your pallas kernel here")


# speedup vs baseline: 2.6829x; 2.6829x over previous
import functools
import jax
import jax.numpy as jnp
from jax import lax
from jax.experimental import pallas as pl
from jax.experimental.pallas import tpu as pltpu
from jax.experimental.pallas import tpu_sc as plsc

T = 2048
D = 1024
H = 4096
E = 8
K = 2
ROW_TILE = 128
P = T * K + E * ROW_TILE
NUM_TILES = P // ROW_TILE
LANES = 128
NB = (T * K) // LANES
NSC = 32
PAIRS_PER_W = (T * K) // NSC
CH = 64


def _ri(v):
    return jnp.floor(v + 0.5)


def _router_dispatch_body(x_ref, rw_ref, meta_ref):
    xf = x_ref[...]
    li = lax.broadcasted_iota(jnp.int32, (T, LANES), 1)
    valid = li < E
    logits = jnp.dot(xf, rw_ref[...], preferred_element_type=jnp.float32)
    neg = jnp.where(valid, logits, -1e30)
    m0 = jnp.max(neg, axis=1, keepdims=True)
    is0 = jnp.logical_and(neg == m0, valid)
    idx0 = jnp.min(jnp.where(is0, li, 999), axis=1, keepdims=True)
    neg2 = jnp.where(li == idx0, -1e30, neg)
    m1 = jnp.max(neg2, axis=1, keepdims=True)
    is1 = jnp.logical_and(neg2 == m1, valid)
    idx1 = jnp.min(jnp.where(is1, li, 999), axis=1, keepdims=True)
    w0 = jax.nn.sigmoid(m0 - m1)

    oh0 = jnp.logical_and(li == idx0, valid).astype(jnp.float32)
    oh1 = jnp.logical_and(li == idx1, valid).astype(jnp.float32)
    oh = jnp.concatenate([oh0, oh1], axis=0)

    counts = jnp.sum(oh, axis=0, keepdims=True)
    pc = jnp.floor((counts + (ROW_TILE - 1)) * (1.0 / ROW_TILE)) * ROW_TILE
    r128 = lax.broadcasted_iota(jnp.int32, (LANES, LANES), 0)
    c128 = lax.broadcasted_iota(jnp.int32, (LANES, LANES), 1)
    upper = (r128 < c128).astype(jnp.float32)
    offs = _ri(jnp.dot(pc, upper, preferred_element_type=jnp.float32,
                       precision=lax.Precision.HIGHEST))

    rS = lax.broadcasted_iota(jnp.int32, (NB, T * K), 0)
    cS = lax.broadcasted_iota(jnp.int32, (NB, T * K), 1) // LANES
    S = (rS == cS).astype(jnp.float32)
    bs = _ri(jnp.dot(S, oh, preferred_element_type=jnp.float32,
                     precision=lax.Precision.HIGHEST))
    r32 = lax.broadcasted_iota(jnp.int32, (NB, NB), 0)
    c32 = lax.broadcasted_iota(jnp.int32, (NB, NB), 1)
    l32 = (r32 > c32).astype(jnp.float32)
    bex = _ri(jnp.dot(l32, bs, preferred_element_type=jnp.float32,
                      precision=lax.Precision.HIGHEST))
    rT = lax.broadcasted_iota(jnp.int32, (T * K, NB), 0) // LANES
    cT = lax.broadcasted_iota(jnp.int32, (T * K, NB), 1)
    St = (rT == cT).astype(jnp.float32)
    bexf = _ri(jnp.dot(St, bex, preferred_element_type=jnp.float32,
                       precision=lax.Precision.HIGHEST))
    lower = (r128 > c128).astype(jnp.float32)
    intra = _ri(jnp.concatenate(
        [jnp.dot(lower, oh[b * LANES:(b + 1) * LANES],
                 preferred_element_type=jnp.float32,
                 precision=lax.Precision.HIGHEST) for b in range(NB)],
        axis=0))
    posf = bexf + intra + offs
    posv = _ri(jnp.sum(posf * oh, axis=1, keepdims=True))
    p0 = posv[:T]
    p1 = posv[T:]

    ti = lax.broadcasted_iota(jnp.int32, (T, 1), 0).astype(jnp.float32)
    cmp = jnp.logical_and(offs <= ti * ROW_TILE, valid).astype(jnp.float32)
    te_col = jnp.sum(cmp, axis=1, keepdims=True) - 1.0
    te_oh = jnp.logical_and(li == (te_col + 0.5).astype(jnp.int32),
                            valid).astype(jnp.float32)
    offs_te = jnp.sum(offs * te_oh, axis=1, keepdims=True)
    cnt_te = jnp.sum(counts * te_oh, axis=1, keepdims=True)
    vc_col = jnp.clip(cnt_te - (ti * ROW_TILE - offs_te), 0.0, 1.0 * ROW_TILE)

    meta = (jnp.where(li == 0, p0, 0.0) + jnp.where(li == 1, p1, 0.0)
            + jnp.where(li == 2, w0, 0.0) + jnp.where(li == 3, 1.0 - w0, 0.0)
            + jnp.where(li == 4, te_col, 0.0) + jnp.where(li == 5, vc_col, 0.0))
    meta_ref[...] = meta


def _gelu(v):
    return 0.5 * v * (1.0 + lax.erf(v * (2.0 ** -0.5)))


def _sc_scatter_body(x_hbm, pos_hbm, xs_hbm, idx_v, rows_v, sem):
    c = lax.axis_index("c")
    s = lax.axis_index("s")
    wid = s * 2 + c
    for sub in range(PAIRS_PER_W // CH):
        pbase = wid * PAIRS_PER_W + sub * CH
        tbase = lax.rem(pbase, T)
        pltpu.sync_copy(pos_hbm.at[pl.ds(pbase, CH)], idx_v)
        pltpu.sync_copy(x_hbm.at[pl.ds(tbase, CH)], rows_v)
        pltpu.async_copy(rows_v, xs_hbm.at[idx_v], sem).wait()


def _ffn1_body(te_ref, vc_ref, xs_ref, w1_ref, hs_ref):
    i = pl.program_id(0)
    vcnt = vc_ref[i]
    ri = lax.broadcasted_iota(jnp.int32, (ROW_TILE, D), 0)
    xb = jnp.where(ri < vcnt, xs_ref[...], 0.0).astype(jnp.bfloat16)
    w1 = w1_ref[0].astype(jnp.bfloat16)
    h = lax.dot_general(xb, w1, (((1,), (1,)), ((), ())),
                        preferred_element_type=jnp.float32)
    hs_ref[...] = _gelu(h).astype(jnp.bfloat16)


def _ffn2_body(te_ref, hs_ref, w2_ref, ys_ref):
    w2 = w2_ref[0].astype(jnp.bfloat16)
    y = lax.dot_general(hs_ref[...], w2, (((1,), (1,)), ((), ())),
                        preferred_element_type=jnp.float32)
    ys_ref[...] = y.astype(jnp.bfloat16)


def _combine_body(meta_ref, ys_ref, out_ref):
    p0 = (meta_ref[:, 0:1] + 0.5).astype(jnp.int32)
    p1 = (meta_ref[:, 1:2] + 0.5).astype(jnp.int32)
    w0 = meta_ref[:, 2:3]
    w1 = meta_ref[:, 3:4]
    rl = lax.broadcasted_iota(jnp.int32, (256, P), 1)
    c = (w0 * (rl == p0).astype(jnp.float32)
         + w1 * (rl == p1).astype(jnp.float32)).astype(jnp.bfloat16)
    out_ref[...] = lax.dot_general(c, ys_ref[...], (((1,), (0,)), ((), ())),
                                   preferred_element_type=jnp.float32)


def _build(interpret=False):
    k0 = pl.pallas_call(
        _router_dispatch_body,
        out_shape=jax.ShapeDtypeStruct((T, LANES), jnp.float32),
        interpret=interpret,
    )
    k1 = pl.pallas_call(
        _ffn1_body,
        grid_spec=pltpu.PrefetchScalarGridSpec(
            num_scalar_prefetch=2,
            grid=(NUM_TILES,),
            in_specs=[
                pl.BlockSpec((ROW_TILE, D), lambda i, te, vc: (i, 0)),
                pl.BlockSpec((1, H, D), lambda i, te, vc: (te[i], 0, 0)),
            ],
            out_specs=pl.BlockSpec((ROW_TILE, H), lambda i, te, vc: (i, 0)),
        ),
        out_shape=jax.ShapeDtypeStruct((P, H), jnp.bfloat16),
        interpret=interpret,
    )
    k2 = pl.pallas_call(
        _ffn2_body,
        grid_spec=pltpu.PrefetchScalarGridSpec(
            num_scalar_prefetch=1,
            grid=(NUM_TILES,),
            in_specs=[
                pl.BlockSpec((ROW_TILE, H), lambda i, te: (i, 0)),
                pl.BlockSpec((1, D, H), lambda i, te: (te[i], 0, 0)),
            ],
            out_specs=pl.BlockSpec((ROW_TILE, D), lambda i, te: (i, 0)),
        ),
        out_shape=jax.ShapeDtypeStruct((P, D), jnp.bfloat16),
        interpret=interpret,
    )
    k3 = pl.pallas_call(
        _combine_body,
        grid=(T // 256,),
        in_specs=[
            pl.BlockSpec((256, LANES), lambda i: (i, 0)),
            pl.BlockSpec((P, D), lambda i: (0, 0)),
        ],
        out_specs=pl.BlockSpec((256, D), lambda i: (i, 0)),
        out_shape=jax.ShapeDtypeStruct((T, D), jnp.float32),
        interpret=interpret,
    )
    return k0, k1, k2, k3


_K0, _K1, _K2, _K3 = _build()


@functools.cache
def _build_s1():
    mesh = plsc.VectorSubcoreMesh(core_axis_name="c", subcore_axis_name="s",
                                  num_cores=2, num_subcores=16)
    return functools.partial(
        pl.kernel, mesh=mesh,
        out_type=jax.ShapeDtypeStruct((P, D), jnp.float32),
        scratch_types=[
            pltpu.VMEM((CH,), jnp.int32),
            pltpu.VMEM((CH, D), jnp.float32),
            pltpu.SemaphoreType.DMA,
        ],
    )(_sc_scatter_body)


@jax.jit
def kernel(x, router_w, fc1_w, fc1_b, fc2_w, fc2_b):
    b, s, d = x.shape
    x_flat = x.reshape(b * s, d)
    rw_pad = jnp.zeros((D, LANES), jnp.float32).at[:, :E].set(router_w.T)
    meta = _K0(x_flat, rw_pad)
    te = (meta[:NUM_TILES, 4] + 0.5).astype(jnp.int32)
    vc = (meta[:NUM_TILES, 5] + 0.5).astype(jnp.int32)
    pos = (jnp.concatenate([meta[:, 0], meta[:, 1]]) + 0.5).astype(jnp.int32)
    xs = _build_s1()(x_flat, pos)
    hs = _K1(te, vc, xs, fc1_w)
    ys = _K2(te, hs, fc2_w)
    out = _K3(meta, ys)
    return out.reshape(b, s, d)
